# baseline (speedup 1.0000x reference)
import functools

import jax
import jax.numpy as jnp
from jax import lax
from jax.experimental import pallas as pl
from jax.experimental.pallas import tpu as pltpu
from jax.experimental.pallas import tpu_sc as plsc

N_U = 10000
N_I = 5000
DIM = 64
TEMP = 0.2
LAMBDA_1 = 0.2
LAMBDA_2 = 1e-07
SVD_Q = 32
BATCH = 4096

BM = 400
GRID_U = N_U // BM
F32 = jnp.float32
_HI = jax.lax.Precision.HIGHEST


def _dot(a, b, precision=None):
    return jax.lax.dot_general(a, b, (((1,), (0,)), ((), ())),
                               precision=precision, preferred_element_type=F32)


def _dot_t0(a, b):
    return jax.lax.dot_general(a, b, (((0,), (0,)), ((), ())),
                               preferred_element_type=F32)


def _dot_t1(a, b):
    return jax.lax.dot_general(a, b, (((1,), (1,)), ((), ())),
                               preferred_element_type=F32)


BF16 = jnp.bfloat16


def _p0_body(a_ref, gt_ref, eu0_ref, ei0t_ref,
             c_ref, raw1_ref, zu1_ref, zi1t_ref, reg_ref):
    y = _dot_t1(a_ref[...], gt_ref[...].astype(BF16))
    zu1_ref[...] = _dot_t1(a_ref[...], ei0t_ref[...].astype(BF16))

    @pl.when(pl.program_id(0) == 0)
    def _():
        c_ref[...] = jnp.zeros_like(c_ref)
        raw1_ref[...] = jnp.zeros_like(raw1_ref)
        zi1t_ref[...] = jnp.zeros_like(zi1t_ref)
        reg_ref[...] = jnp.reshape(
            jnp.sum(ei0t_ref[...] * ei0t_ref[...]), (1, 1))

    c_ref[...] += _dot_t0(y, y)
    raw1_ref[...] += _dot_t0(y.astype(BF16), a_ref[...])
    zi1t_ref[...] += _dot_t0(eu0_ref[...].astype(BF16), a_ref[...])
    reg_ref[...] += jnp.reshape(jnp.sum(eu0_ref[...] * eu0_ref[...]), (1, 1))


def _pass_a(a16, gt, eu0, ei0t):
    kq = gt.shape[0]
    return pl.pallas_call(
        _p0_body,
        grid=(GRID_U,),
        in_specs=[pl.BlockSpec((BM, N_I), lambda i: (i, 0)),
                  pl.BlockSpec((kq, N_I), lambda i: (0, 0)),
                  pl.BlockSpec((BM, DIM), lambda i: (i, 0)),
                  pl.BlockSpec((DIM, N_I), lambda i: (0, 0))],
        out_specs=[pl.BlockSpec((kq, kq), lambda i: (0, 0)),
                   pl.BlockSpec((kq, N_I), lambda i: (0, 0)),
                   pl.BlockSpec((BM, DIM), lambda i: (i, 0)),
                   pl.BlockSpec((DIM, N_I), lambda i: (0, 0)),
                   pl.BlockSpec((1, 1), lambda i: (0, 0))],
        out_shape=[jax.ShapeDtypeStruct((kq, kq), F32),
                   jax.ShapeDtypeStruct((kq, N_I), F32),
                   jax.ShapeDtypeStruct((N_U, DIM), F32),
                   jax.ShapeDtypeStruct((DIM, N_I), F32),
                   jax.ShapeDtypeStruct((1, 1), F32)],
    )(a16, gt, eu0, ei0t)


def _eye(q):
    ri = jax.lax.broadcasted_iota(jnp.int32, (q, q), 0)
    ci = jax.lax.broadcasted_iota(jnp.int32, (q, q), 1)
    return jnp.where(ri == ci, 1.0, 0.0).astype(F32)


def _trace(C):
    q = C.shape[0]
    ri = jax.lax.broadcasted_iota(jnp.int32, (q, q), 0)
    ci = jax.lax.broadcasted_iota(jnp.int32, (q, q), 1)
    return jnp.sum(jnp.where(ri == ci, C, 0.0))


def _ns_invsqrt(C, iters=20):
    eye = _eye(C.shape[0])
    s = _trace(C)
    y = C * (1.0 / s)
    z = eye
    for _ in range(iters):
        t = 1.5 * eye - 0.5 * _dot(z, y, precision=_HI)
        y = _dot(y, t, precision=_HI)
        z = _dot(t, z, precision=_HI)
    return z * jax.lax.rsqrt(s)


def _ns_inv(C, iters=20):
    eye = _eye(C.shape[0])
    x = eye * (1.0 / _trace(C))
    for _ in range(iters):
        x = _dot(x, 2.0 * eye - _dot(C, x, precision=_HI), precision=_HI)
    return x


def _orth_chain(c_prev, raw):
    wt = _dot(_ns_invsqrt(c_prev), raw, precision=_HI)
    s = _ns_invsqrt(_dot_t1(wt, wt))
    return _dot(s, wt, precision=_HI)


def _pb_body(a_ref, c0_ref, raw1_ref, c_ref, raw2_ref, zt_ref):
    @pl.when(pl.program_id(0) == 0)
    def _():
        zt_ref[...] = _orth_chain(c0_ref[...], raw1_ref[...]).astype(BF16)
        c_ref[...] = jnp.zeros_like(c_ref)
        raw2_ref[...] = jnp.zeros_like(raw2_ref)

    y = _dot_t1(a_ref[...], zt_ref[...])
    c_ref[...] += _dot_t0(y, y)
    raw2_ref[...] += _dot_t0(y.astype(BF16), a_ref[...])


def _pass_b(a16, c0, raw1):
    kq = SVD_Q
    return pl.pallas_call(
        _pb_body,
        grid=(GRID_U,),
        in_specs=[pl.BlockSpec((BM, N_I), lambda i: (i, 0)),
                  pl.BlockSpec((kq, kq), lambda i: (0, 0)),
                  pl.BlockSpec((kq, N_I), lambda i: (0, 0))],
        out_specs=[pl.BlockSpec((kq, kq), lambda i: (0, 0)),
                   pl.BlockSpec((kq, N_I), lambda i: (0, 0))],
        out_shape=[jax.ShapeDtypeStruct((kq, kq), F32),
                   jax.ShapeDtypeStruct((kq, N_I), F32)],
        scratch_shapes=[pltpu.VMEM((kq, N_I), BF16)],
    )(a16, c0, raw1)


def _pc_body(a_ref, c1_ref, raw2_ref, zu1_ref, zi1t_ref, eu0_ref, ei0t_ref,
             y_ref, c_ref, bt_ref, eu_ref, eit_ref, t2t_ref, zt_ref):
    @pl.when(pl.program_id(0) == 0)
    def _():
        zt_ref[...] = _orth_chain(c1_ref[...], raw2_ref[...]).astype(BF16)
        c_ref[...] = jnp.zeros_like(c_ref)
        bt_ref[...] = jnp.zeros_like(bt_ref)
        eit_ref[...] = ei0t_ref[...] + zi1t_ref[...]
        t2t_ref[...] = jnp.zeros_like(t2t_ref)

    y = _dot_t1(a_ref[...], zt_ref[...])
    y_ref[...] = y
    zu1 = zu1_ref[...]
    eu0 = eu0_ref[...]
    eu_ref[...] = eu0 + zu1 + _dot_t1(a_ref[...], zi1t_ref[...].astype(BF16))

    c_ref[...] += _dot_t0(y, y)
    bt_ref[...] += _dot_t0(y.astype(BF16), a_ref[...])
    eit_ref[...] += _dot_t0(zu1.astype(BF16), a_ref[...])
    t2t_ref[...] += _dot_t0(eu0 + zu1, y)


def _pass_c(a16, c1, raw2, zu1, zi1t, eu0, ei0t):
    return pl.pallas_call(
        _pc_body,
        grid=(GRID_U,),
        in_specs=[pl.BlockSpec((BM, N_I), lambda i: (i, 0)),
                  pl.BlockSpec((SVD_Q, SVD_Q), lambda i: (0, 0)),
                  pl.BlockSpec((SVD_Q, N_I), lambda i: (0, 0)),
                  pl.BlockSpec((BM, DIM), lambda i: (i, 0)),
                  pl.BlockSpec((DIM, N_I), lambda i: (0, 0)),
                  pl.BlockSpec((BM, DIM), lambda i: (i, 0)),
                  pl.BlockSpec((DIM, N_I), lambda i: (0, 0))],
        out_specs=[pl.BlockSpec((BM, SVD_Q), lambda i: (i, 0)),
                   pl.BlockSpec((SVD_Q, SVD_Q), lambda i: (0, 0)),
                   pl.BlockSpec((SVD_Q, N_I), lambda i: (0, 0)),
                   pl.BlockSpec((BM, DIM), lambda i: (i, 0)),
                   pl.BlockSpec((DIM, N_I), lambda i: (0, 0)),
                   pl.BlockSpec((DIM, SVD_Q), lambda i: (0, 0))],
        out_shape=[jax.ShapeDtypeStruct((N_U, SVD_Q), F32),
                   jax.ShapeDtypeStruct((SVD_Q, SVD_Q), F32),
                   jax.ShapeDtypeStruct((SVD_Q, N_I), F32),
                   jax.ShapeDtypeStruct((N_U, DIM), F32),
                   jax.ShapeDtypeStruct((DIM, N_I), F32),
                   jax.ShapeDtypeStruct((DIM, SVD_Q), F32)],
        scratch_shapes=[pltpu.VMEM((SVD_Q, N_I), BF16)],
    )(a16, c1, raw2, zu1, zi1t, eu0, ei0t)


def _guei_body(y_ref, c_ref, bt_ref, t2t_ref, eu0_ref, eu_ref,
               ei0t_ref, zi1t_ref, eit_ref, ueu_ref, uei_ref, t1_ref):
    @pl.when(pl.program_id(0) == 0)
    def _():
        m = _ns_inv(c_ref[...])
        t1_ref[...] = _dot(
            m, _dot_t1(bt_ref[...], ei0t_ref[...] + zi1t_ref[...]))
        git = ei0t_ref[...] + _dot(_dot(t2t_ref[...], m), bt_ref[...])
        uei_ref[...] = jnp.concatenate([git, eit_ref[...]], axis=0)

    ueu_ref[...] = jnp.concatenate(
        [eu0_ref[...] + _dot(y_ref[...], t1_ref[...]), eu_ref[...]], axis=1)


def _guei(y2, c4, bt, t2t, eu0, e_u, ei0t, zi1t, eit):
    return pl.pallas_call(
        _guei_body,
        grid=(GRID_U,),
        in_specs=[pl.BlockSpec((BM, SVD_Q), lambda i: (i, 0)),
                  pl.BlockSpec((SVD_Q, SVD_Q), lambda i: (0, 0)),
                  pl.BlockSpec((SVD_Q, N_I), lambda i: (0, 0)),
                  pl.BlockSpec((DIM, SVD_Q), lambda i: (0, 0)),
                  pl.BlockSpec((BM, DIM), lambda i: (i, 0)),
                  pl.BlockSpec((BM, DIM), lambda i: (i, 0)),
                  pl.BlockSpec((DIM, N_I), lambda i: (0, 0)),
                  pl.BlockSpec((DIM, N_I), lambda i: (0, 0)),
                  pl.BlockSpec((DIM, N_I), lambda i: (0, 0))],
        out_specs=[pl.BlockSpec((BM, 2 * DIM), lambda i: (i, 0)),
                   pl.BlockSpec((2 * DIM, N_I), lambda i: (0, 0))],
        out_shape=[jax.ShapeDtypeStruct((N_U, 2 * DIM), F32),
                   jax.ShapeDtypeStruct((2 * DIM, N_I), F32)],
        scratch_shapes=[pltpu.VMEM((SVD_Q, DIM), F32)],
    )(y2, c4, bt, t2t, eu0, e_u, ei0t, zi1t, eit)


def _sc_gather_all(ue_u, ue_i, row_ids, col_ids, pos, neg):
    info = plsc.get_sparse_core_info()
    nc, ns = info.num_cores, info.num_subcores
    nw = nc * ns
    bpw = BATCH // nw
    mesh = plsc.VectorSubcoreMesh(core_axis_name="c", subcore_axis_name="s")
    out = jax.ShapeDtypeStruct((BATCH, 2 * DIM), F32)

    @functools.partial(
        pl.kernel, mesh=mesh,
        out_type=(out,) * 4,
        scratch_types=[pltpu.VMEM((bpw,), jnp.int32),
                       pltpu.VMEM((bpw, 2 * DIM), F32),
                       pltpu.SemaphoreType.DMA],
    )
    def k(tu_hbm, ti_hbm, rid_hbm, cid_hbm, pos_hbm, neg_hbm,
          o_ru, o_rc, o_rp, o_rn, idx_v, rows_v, sem):
        wid = lax.axis_index("s") * nc + lax.axis_index("c")
        base = wid * bpw

        def gather(idx_hbm, table_hbm, out_hbm):
            pltpu.sync_copy(idx_hbm.at[pl.ds(base, bpw)], idx_v)
            pltpu.async_copy(table_hbm.at[idx_v], rows_v, sem).wait()
            pltpu.sync_copy(rows_v, out_hbm.at[pl.ds(base, bpw)])

        gather(rid_hbm, tu_hbm, o_ru)
        gather(cid_hbm, ti_hbm, o_rc)
        gather(pos_hbm, ti_hbm, o_rp)
        gather(neg_hbm, ti_hbm, o_rn)

    return k(ue_u, ue_i, row_ids, col_ids, pos, neg)


_BB = 512
_NB = BATCH // _BB
_UCH = 2000
_ICH = 2500


def _loss_body(ueu_ref, eit_ref, reg_ref, ru_ref, rc_ref, rp_ref, rn_ref,
               loss_ref, oth_ref, acc_ref):
    i = pl.program_id(0)
    gur, eur = ru_ref[:, :DIM], ru_ref[:, DIM:]
    gic, eic = rc_ref[:, :DIM], rc_ref[:, DIM:]
    eip, ein = rp_ref[:, DIM:], rn_ref[:, DIM:]

    su = jnp.zeros((_BB, 1), F32)
    for kc in range(N_U // _UCH):
        logits = _dot_t1(gur, ueu_ref[kc * _UCH:(kc + 1) * _UCH, DIM:])
        su = su + jnp.sum(jnp.exp(logits * (1.0 / TEMP)), axis=1, keepdims=True)
    si = jnp.zeros((_BB, 1), F32)
    for kc in range(N_I // _ICH):
        logits = _dot(gic, eit_ref[:, kc * _ICH:(kc + 1) * _ICH])
        si = si + jnp.sum(jnp.exp(logits * (1.0 / TEMP)), axis=1, keepdims=True)
    nl_u = jnp.sum(jnp.log(su + 1e-08))
    nl_i = jnp.sum(jnp.log(si + 1e-08))

    pu = jnp.sum(gur * eur, axis=1, keepdims=True) * (1.0 / TEMP)
    pi = jnp.sum(gic * eic, axis=1, keepdims=True) * (1.0 / TEMP)
    pos = jnp.sum(jnp.clip(pu, -5.0, 5.0)) + jnp.sum(jnp.clip(pi, -5.0, 5.0))
    d = jnp.sum(eur * eip, axis=1, keepdims=True) - \
        jnp.sum(eur * ein, axis=1, keepdims=True)
    bpr = jnp.sum(jnp.log(1.0 + jnp.exp(-d)))

    upd = jnp.concatenate(
        [jnp.reshape(nl_u, (1, 1)), jnp.reshape(nl_i, (1, 1)),
         jnp.reshape(pos, (1, 1)), jnp.reshape(bpr, (1, 1)),
         jnp.zeros((1, 124), F32)], axis=1)

    @pl.when(i == 0)
    def _():
        acc_ref[...] = jnp.zeros_like(acc_ref)
    acc_ref[...] += upd

    @pl.when(i == _NB - 1)
    def _():
        acc = acc_ref[...]
        inv_b = 1.0 / BATCH
        neg_score = (acc[0, 0] + acc[0, 1]) * inv_b
        pos_score = acc[0, 2] * inv_b
        loss_bpr = acc[0, 3] * inv_b
        loss_cl = -pos_score + neg_score
        loss = loss_bpr + LAMBDA_1 * loss_cl + LAMBDA_2 * reg_ref[0, 0]
        loss_ref[...] = jnp.reshape(loss, (1, 1))
        oth_ref[...] = jnp.concatenate(
            [jnp.full((1, 1), loss_bpr, F32),
             jnp.full((1, 1), LAMBDA_1 * loss_cl, F32)], axis=1)


def _loss(ueu, eit, reg, ru, rc, rp, rn):
    bspec = pl.BlockSpec((_BB, 2 * DIM), lambda i: (i, 0))
    return pl.pallas_call(
        _loss_body,
        grid=(_NB,),
        in_specs=[pl.BlockSpec((N_U, 2 * DIM), lambda i: (0, 0)),
                  pl.BlockSpec((DIM, N_I), lambda i: (0, 0)),
                  pl.BlockSpec((1, 1), lambda i: (0, 0)),
                  bspec, bspec, bspec, bspec],
        out_specs=[pl.BlockSpec((1, 1), lambda i: (0, 0)),
                   pl.BlockSpec((1, 2), lambda i: (0, 0))],
        out_shape=[jax.ShapeDtypeStruct((1, 1), F32),
                   jax.ShapeDtypeStruct((1, 2), F32)],
        scratch_shapes=[pltpu.VMEM((1, 128), F32)],
    )(ueu, eit, reg, ru, rc, rp, rn)


def kernel(adj, row_ids, col_ids, pos, neg, E_u_0, E_i_0):
    g0t = jax.random.normal(jax.random.key(42), (N_I, SVD_Q), dtype=F32).T
    ei0t = E_i_0.T

    a16 = adj.astype(BF16)
    c0, raw1, zu1, zi1t, reg = _pass_a(a16, g0t, E_u_0, ei0t)
    c1, raw2 = _pass_b(a16, c0, raw1)
    y2, c4, bt, e_u, e_it, t2t = _pass_c(a16, c1, raw2, zu1, zi1t,
                                         E_u_0, ei0t)
    ue_u, uei_t = _guei(y2, c4, bt, t2t, E_u_0, e_u, ei0t, zi1t, e_it)
    ue_i = uei_t.T

    ru, rc, rp, rn = _sc_gather_all(ue_u, ue_i, row_ids, col_ids, pos, neg)

    loss, oth = _loss(ue_u, e_it, reg, ru, rc, rp, rn)
    return loss[0, 0], oth[0]


# speedup vs baseline: 1.0297x; 1.0297x over previous
import functools

import jax
import jax.numpy as jnp
from jax import lax
from jax.experimental import pallas as pl
from jax.experimental.pallas import tpu as pltpu
from jax.experimental.pallas import tpu_sc as plsc

N_U = 10000
N_I = 5000
DIM = 64
TEMP = 0.2
LAMBDA_1 = 0.2
LAMBDA_2 = 1e-07
SVD_Q = 32
BATCH = 4096

BM = 1000
GRID_U = N_U // BM
F32 = jnp.float32
_HI = jax.lax.Precision.HIGHEST


def _dot(a, b, precision=None):
    return jax.lax.dot_general(a, b, (((1,), (0,)), ((), ())),
                               precision=precision, preferred_element_type=F32)


def _dot_t0(a, b):
    return jax.lax.dot_general(a, b, (((0,), (0,)), ((), ())),
                               preferred_element_type=F32)


def _dot_t1(a, b):
    return jax.lax.dot_general(a, b, (((1,), (1,)), ((), ())),
                               preferred_element_type=F32)


BF16 = jnp.bfloat16


def _p0_body(a_ref, gt_ref, eu0_ref, ei0t_ref,
             c_ref, raw1_ref, zu1_ref, zi1t_ref, reg_ref):
    y = _dot_t1(a_ref[...], gt_ref[...].astype(BF16))
    zu1_ref[...] = _dot_t1(a_ref[...], ei0t_ref[...].astype(BF16))

    @pl.when(pl.program_id(0) == 0)
    def _():
        c_ref[...] = jnp.zeros_like(c_ref)
        raw1_ref[...] = jnp.zeros_like(raw1_ref)
        zi1t_ref[...] = jnp.zeros_like(zi1t_ref)
        reg_ref[...] = jnp.reshape(
            jnp.sum(ei0t_ref[...] * ei0t_ref[...]), (1, 1))

    c_ref[...] += _dot_t0(y, y)
    raw1_ref[...] += _dot_t0(y.astype(BF16), a_ref[...])
    zi1t_ref[...] += _dot_t0(eu0_ref[...].astype(BF16), a_ref[...])
    reg_ref[...] += jnp.reshape(jnp.sum(eu0_ref[...] * eu0_ref[...]), (1, 1))


def _pass_a(a16, gt, eu0, ei0t):
    kq = gt.shape[0]
    return pl.pallas_call(
        _p0_body,
        grid=(GRID_U,),
        in_specs=[pl.BlockSpec((BM, N_I), lambda i: (i, 0)),
                  pl.BlockSpec((kq, N_I), lambda i: (0, 0)),
                  pl.BlockSpec((BM, DIM), lambda i: (i, 0)),
                  pl.BlockSpec((DIM, N_I), lambda i: (0, 0))],
        out_specs=[pl.BlockSpec((kq, kq), lambda i: (0, 0)),
                   pl.BlockSpec((kq, N_I), lambda i: (0, 0)),
                   pl.BlockSpec((BM, DIM), lambda i: (i, 0)),
                   pl.BlockSpec((DIM, N_I), lambda i: (0, 0)),
                   pl.BlockSpec((1, 1), lambda i: (0, 0))],
        out_shape=[jax.ShapeDtypeStruct((kq, kq), F32),
                   jax.ShapeDtypeStruct((kq, N_I), F32),
                   jax.ShapeDtypeStruct((N_U, DIM), F32),
                   jax.ShapeDtypeStruct((DIM, N_I), F32),
                   jax.ShapeDtypeStruct((1, 1), F32)],
    )(a16, gt, eu0, ei0t)


def _eye(q):
    ri = jax.lax.broadcasted_iota(jnp.int32, (q, q), 0)
    ci = jax.lax.broadcasted_iota(jnp.int32, (q, q), 1)
    return jnp.where(ri == ci, 1.0, 0.0).astype(F32)


def _trace(C):
    q = C.shape[0]
    ri = jax.lax.broadcasted_iota(jnp.int32, (q, q), 0)
    ci = jax.lax.broadcasted_iota(jnp.int32, (q, q), 1)
    return jnp.sum(jnp.where(ri == ci, C, 0.0))


def _ns_invsqrt(C, iters=12):
    eye = _eye(C.shape[0])
    s = _trace(C)
    y = C * (1.0 / s)
    z = eye
    for _ in range(iters):
        t = 1.5 * eye - 0.5 * _dot(z, y, precision=_HI)
        y = _dot(y, t, precision=_HI)
        z = _dot(t, z, precision=_HI)
    return z * jax.lax.rsqrt(s)


def _ns_inv(C, iters=20):
    eye = _eye(C.shape[0])
    x = eye * (1.0 / _trace(C))
    for _ in range(iters):
        x = _dot(x, 2.0 * eye - _dot(C, x, precision=_HI), precision=_HI)
    return x


def _orth_chain(c_prev, raw):
    wt = _dot(_ns_invsqrt(c_prev), raw, precision=_HI)
    s = _ns_invsqrt(_dot_t1(wt, wt))
    return _dot(s, wt, precision=_HI)


def _pb_body(a_ref, c0_ref, raw1_ref, c_ref, raw2_ref, zt_ref):
    @pl.when(pl.program_id(0) == 0)
    def _():
        zt_ref[...] = _orth_chain(c0_ref[...], raw1_ref[...]).astype(BF16)
        c_ref[...] = jnp.zeros_like(c_ref)
        raw2_ref[...] = jnp.zeros_like(raw2_ref)

    y = _dot_t1(a_ref[...], zt_ref[...])
    c_ref[...] += _dot_t0(y, y)
    raw2_ref[...] += _dot_t0(y.astype(BF16), a_ref[...])


def _pass_b(a16, c0, raw1):
    kq = SVD_Q
    return pl.pallas_call(
        _pb_body,
        grid=(GRID_U,),
        in_specs=[pl.BlockSpec((BM, N_I), lambda i: (i, 0)),
                  pl.BlockSpec((kq, kq), lambda i: (0, 0)),
                  pl.BlockSpec((kq, N_I), lambda i: (0, 0))],
        out_specs=[pl.BlockSpec((kq, kq), lambda i: (0, 0)),
                   pl.BlockSpec((kq, N_I), lambda i: (0, 0))],
        out_shape=[jax.ShapeDtypeStruct((kq, kq), F32),
                   jax.ShapeDtypeStruct((kq, N_I), F32)],
        scratch_shapes=[pltpu.VMEM((kq, N_I), BF16)],
    )(a16, c0, raw1)


def _pc_body(a_ref, c1_ref, raw2_ref, zu1_ref, zi1t_ref, eu0_ref, ei0t_ref,
             y_ref, c_ref, bt_ref, eu_ref, eit_ref, t2t_ref, zt_ref):
    @pl.when(pl.program_id(0) == 0)
    def _():
        zt_ref[...] = _orth_chain(c1_ref[...], raw2_ref[...]).astype(BF16)
        c_ref[...] = jnp.zeros_like(c_ref)
        bt_ref[...] = jnp.zeros_like(bt_ref)
        eit_ref[...] = ei0t_ref[...] + zi1t_ref[...]
        t2t_ref[...] = jnp.zeros_like(t2t_ref)

    y = _dot_t1(a_ref[...], zt_ref[...])
    y_ref[...] = y
    zu1 = zu1_ref[...]
    eu0 = eu0_ref[...]
    eu_ref[...] = eu0 + zu1 + _dot_t1(a_ref[...], zi1t_ref[...].astype(BF16))

    c_ref[...] += _dot_t0(y, y)
    bt_ref[...] += _dot_t0(y.astype(BF16), a_ref[...])
    eit_ref[...] += _dot_t0(zu1.astype(BF16), a_ref[...])
    t2t_ref[...] += _dot_t0(eu0 + zu1, y)


def _pass_c(a16, c1, raw2, zu1, zi1t, eu0, ei0t):
    return pl.pallas_call(
        _pc_body,
        grid=(GRID_U,),
        in_specs=[pl.BlockSpec((BM, N_I), lambda i: (i, 0)),
                  pl.BlockSpec((SVD_Q, SVD_Q), lambda i: (0, 0)),
                  pl.BlockSpec((SVD_Q, N_I), lambda i: (0, 0)),
                  pl.BlockSpec((BM, DIM), lambda i: (i, 0)),
                  pl.BlockSpec((DIM, N_I), lambda i: (0, 0)),
                  pl.BlockSpec((BM, DIM), lambda i: (i, 0)),
                  pl.BlockSpec((DIM, N_I), lambda i: (0, 0))],
        out_specs=[pl.BlockSpec((BM, SVD_Q), lambda i: (i, 0)),
                   pl.BlockSpec((SVD_Q, SVD_Q), lambda i: (0, 0)),
                   pl.BlockSpec((SVD_Q, N_I), lambda i: (0, 0)),
                   pl.BlockSpec((BM, DIM), lambda i: (i, 0)),
                   pl.BlockSpec((DIM, N_I), lambda i: (0, 0)),
                   pl.BlockSpec((DIM, SVD_Q), lambda i: (0, 0))],
        out_shape=[jax.ShapeDtypeStruct((N_U, SVD_Q), F32),
                   jax.ShapeDtypeStruct((SVD_Q, SVD_Q), F32),
                   jax.ShapeDtypeStruct((SVD_Q, N_I), F32),
                   jax.ShapeDtypeStruct((N_U, DIM), F32),
                   jax.ShapeDtypeStruct((DIM, N_I), F32),
                   jax.ShapeDtypeStruct((DIM, SVD_Q), F32)],
        scratch_shapes=[pltpu.VMEM((SVD_Q, N_I), BF16)],
    )(a16, c1, raw2, zu1, zi1t, eu0, ei0t)


def _guei_body(y_ref, c_ref, bt_ref, t2t_ref, eu0_ref, eu_ref,
               ei0t_ref, zi1t_ref, eit_ref, ueu_ref, uei_ref, t1_ref):
    @pl.when(pl.program_id(0) == 0)
    def _():
        m = _ns_inv(c_ref[...])
        t1_ref[...] = _dot(
            m, _dot_t1(bt_ref[...], ei0t_ref[...] + zi1t_ref[...]))
        git = ei0t_ref[...] + _dot(_dot(t2t_ref[...], m), bt_ref[...])
        uei_ref[...] = jnp.concatenate([git, eit_ref[...]], axis=0)

    ueu_ref[...] = jnp.concatenate(
        [eu0_ref[...] + _dot(y_ref[...], t1_ref[...]), eu_ref[...]], axis=1)


def _guei(y2, c4, bt, t2t, eu0, e_u, ei0t, zi1t, eit):
    return pl.pallas_call(
        _guei_body,
        grid=(GRID_U,),
        in_specs=[pl.BlockSpec((BM, SVD_Q), lambda i: (i, 0)),
                  pl.BlockSpec((SVD_Q, SVD_Q), lambda i: (0, 0)),
                  pl.BlockSpec((SVD_Q, N_I), lambda i: (0, 0)),
                  pl.BlockSpec((DIM, SVD_Q), lambda i: (0, 0)),
                  pl.BlockSpec((BM, DIM), lambda i: (i, 0)),
                  pl.BlockSpec((BM, DIM), lambda i: (i, 0)),
                  pl.BlockSpec((DIM, N_I), lambda i: (0, 0)),
                  pl.BlockSpec((DIM, N_I), lambda i: (0, 0)),
                  pl.BlockSpec((DIM, N_I), lambda i: (0, 0))],
        out_specs=[pl.BlockSpec((BM, 2 * DIM), lambda i: (i, 0)),
                   pl.BlockSpec((2 * DIM, N_I), lambda i: (0, 0))],
        out_shape=[jax.ShapeDtypeStruct((N_U, 2 * DIM), F32),
                   jax.ShapeDtypeStruct((2 * DIM, N_I), F32)],
        scratch_shapes=[pltpu.VMEM((SVD_Q, DIM), F32)],
    )(y2, c4, bt, t2t, eu0, e_u, ei0t, zi1t, eit)


def _sc_gather_all(ue_u, ue_i, row_ids, col_ids, pos, neg):
    info = plsc.get_sparse_core_info()
    nc, ns = info.num_cores, info.num_subcores
    nw = nc * ns
    bpw = BATCH // nw
    mesh = plsc.VectorSubcoreMesh(core_axis_name="c", subcore_axis_name="s")
    out = jax.ShapeDtypeStruct((BATCH, 2 * DIM), F32)

    @functools.partial(
        pl.kernel, mesh=mesh,
        out_type=(out,) * 4,
        scratch_types=[pltpu.VMEM((bpw,), jnp.int32),
                       pltpu.VMEM((bpw, 2 * DIM), F32),
                       pltpu.SemaphoreType.DMA],
    )
    def k(tu_hbm, ti_hbm, rid_hbm, cid_hbm, pos_hbm, neg_hbm,
          o_ru, o_rc, o_rp, o_rn, idx_v, rows_v, sem):
        wid = lax.axis_index("s") * nc + lax.axis_index("c")
        base = wid * bpw

        def gather(idx_hbm, table_hbm, out_hbm):
            pltpu.sync_copy(idx_hbm.at[pl.ds(base, bpw)], idx_v)
            pltpu.async_copy(table_hbm.at[idx_v], rows_v, sem).wait()
            pltpu.sync_copy(rows_v, out_hbm.at[pl.ds(base, bpw)])

        gather(rid_hbm, tu_hbm, o_ru)
        gather(cid_hbm, ti_hbm, o_rc)
        gather(pos_hbm, ti_hbm, o_rp)
        gather(neg_hbm, ti_hbm, o_rn)

    return k(ue_u, ue_i, row_ids, col_ids, pos, neg)


_BB = 512
_NB = BATCH // _BB
_UCH = 2000
_ICH = 2500


def _loss_body(ueu_ref, eit_ref, reg_ref, ru_ref, rc_ref, rp_ref, rn_ref,
               loss_ref, oth_ref, acc_ref):
    i = pl.program_id(0)
    gur, eur = ru_ref[:, :DIM], ru_ref[:, DIM:]
    gic, eic = rc_ref[:, :DIM], rc_ref[:, DIM:]
    eip, ein = rp_ref[:, DIM:], rn_ref[:, DIM:]

    su = jnp.zeros((_BB, 1), F32)
    for kc in range(N_U // _UCH):
        logits = _dot_t1(gur, ueu_ref[kc * _UCH:(kc + 1) * _UCH, DIM:])
        su = su + jnp.sum(jnp.exp(logits * (1.0 / TEMP)), axis=1, keepdims=True)
    si = jnp.zeros((_BB, 1), F32)
    for kc in range(N_I // _ICH):
        logits = _dot(gic, eit_ref[:, kc * _ICH:(kc + 1) * _ICH])
        si = si + jnp.sum(jnp.exp(logits * (1.0 / TEMP)), axis=1, keepdims=True)
    nl_u = jnp.sum(jnp.log(su + 1e-08))
    nl_i = jnp.sum(jnp.log(si + 1e-08))

    pu = jnp.sum(gur * eur, axis=1, keepdims=True) * (1.0 / TEMP)
    pi = jnp.sum(gic * eic, axis=1, keepdims=True) * (1.0 / TEMP)
    pos = jnp.sum(jnp.clip(pu, -5.0, 5.0)) + jnp.sum(jnp.clip(pi, -5.0, 5.0))
    d = jnp.sum(eur * eip, axis=1, keepdims=True) - \
        jnp.sum(eur * ein, axis=1, keepdims=True)
    bpr = jnp.sum(jnp.log(1.0 + jnp.exp(-d)))

    upd = jnp.concatenate(
        [jnp.reshape(nl_u, (1, 1)), jnp.reshape(nl_i, (1, 1)),
         jnp.reshape(pos, (1, 1)), jnp.reshape(bpr, (1, 1)),
         jnp.zeros((1, 124), F32)], axis=1)

    @pl.when(i == 0)
    def _():
        acc_ref[...] = jnp.zeros_like(acc_ref)
    acc_ref[...] += upd

    @pl.when(i == _NB - 1)
    def _():
        acc = acc_ref[...]
        inv_b = 1.0 / BATCH
        neg_score = (acc[0, 0] + acc[0, 1]) * inv_b
        pos_score = acc[0, 2] * inv_b
        loss_bpr = acc[0, 3] * inv_b
        loss_cl = -pos_score + neg_score
        loss = loss_bpr + LAMBDA_1 * loss_cl + LAMBDA_2 * reg_ref[0, 0]
        loss_ref[...] = jnp.reshape(loss, (1, 1))
        oth_ref[...] = jnp.concatenate(
            [jnp.full((1, 1), loss_bpr, F32),
             jnp.full((1, 1), LAMBDA_1 * loss_cl, F32)], axis=1)


def _loss(ueu, eit, reg, ru, rc, rp, rn):
    bspec = pl.BlockSpec((_BB, 2 * DIM), lambda i: (i, 0))
    return pl.pallas_call(
        _loss_body,
        grid=(_NB,),
        in_specs=[pl.BlockSpec((N_U, 2 * DIM), lambda i: (0, 0)),
                  pl.BlockSpec((DIM, N_I), lambda i: (0, 0)),
                  pl.BlockSpec((1, 1), lambda i: (0, 0)),
                  bspec, bspec, bspec, bspec],
        out_specs=[pl.BlockSpec((1, 1), lambda i: (0, 0)),
                   pl.BlockSpec((1, 2), lambda i: (0, 0))],
        out_shape=[jax.ShapeDtypeStruct((1, 1), F32),
                   jax.ShapeDtypeStruct((1, 2), F32)],
        scratch_shapes=[pltpu.VMEM((1, 128), F32)],
    )(ueu, eit, reg, ru, rc, rp, rn)


def kernel(adj, row_ids, col_ids, pos, neg, E_u_0, E_i_0):
    g0t = jax.random.normal(jax.random.key(42), (N_I, SVD_Q), dtype=F32).T
    ei0t = E_i_0.T

    a16 = adj.astype(BF16)
    c0, raw1, zu1, zi1t, reg = _pass_a(a16, g0t, E_u_0, ei0t)
    c1, raw2 = _pass_b(a16, c0, raw1)
    y2, c4, bt, e_u, e_it, t2t = _pass_c(a16, c1, raw2, zu1, zi1t,
                                         E_u_0, ei0t)
    ue_u, uei_t = _guei(y2, c4, bt, t2t, E_u_0, e_u, ei0t, zi1t, e_it)
    ue_i = uei_t.T

    ru, rc, rp, rn = _sc_gather_all(ue_u, ue_i, row_ids, col_ids, pos, neg)

    loss, oth = _loss(ue_u, e_it, reg, ru, rc, rp, rn)
    return loss[0, 0], oth[0]
